# baseline (speedup 1.0000x reference)
import jax
import jax.numpy as jnp
from jax.experimental import pallas as pl
from jax.experimental.pallas import tpu as pltpu

_BI = 400


def _h1_body(feat_ref, w1_ref, out_ref):
    out_ref[...] = jnp.dot(feat_ref[...], w1_ref[...],
                           preferred_element_type=jnp.float32)


def _layer1_body(adj_ref, h1_ref, b1_ref, w2_ref, x1_ref, g2_ref):
    acc = jnp.dot(adj_ref[...], h1_ref[...],
                  preferred_element_type=jnp.float32)
    x1 = jnp.maximum(acc + b1_ref[...], 0.0)
    x1_ref[...] = x1
    g2_ref[...] = jnp.dot(x1, w2_ref[...],
                          preferred_element_type=jnp.float32)


def _layer2_body(adj_ref, g2_ref, b2_ref, out_ref):
    acc = jnp.dot(adj_ref[...], g2_ref[...],
                  preferred_element_type=jnp.float32) + b2_ref[...]
    m = jnp.max(acc, axis=1, keepdims=True)
    s = acc - m
    lse = jnp.log(jnp.sum(jnp.exp(s), axis=1, keepdims=True))
    out_ref[...] = s - lse


def kernel(feature, adj, W1, b1, W2, b2):
    n, f_in = feature.shape
    hid = W1.shape[1]
    c = W2.shape[1]
    b1r = b1.reshape(1, hid)
    b2r = b2.reshape(1, c)

    h1 = pl.pallas_call(
        _h1_body,
        out_shape=jax.ShapeDtypeStruct((n, hid), jnp.float32),
    )(feature, W1)

    grid = (n // _BI,)
    x1, g2 = pl.pallas_call(
        _layer1_body,
        grid=grid,
        in_specs=[
            pl.BlockSpec((_BI, n), lambda i: (i, 0)),
            pl.BlockSpec((n, hid), lambda i: (0, 0)),
            pl.BlockSpec((1, hid), lambda i: (0, 0)),
            pl.BlockSpec((hid, c), lambda i: (0, 0)),
        ],
        out_specs=[
            pl.BlockSpec((_BI, hid), lambda i: (i, 0)),
            pl.BlockSpec((_BI, c), lambda i: (i, 0)),
        ],
        out_shape=[
            jax.ShapeDtypeStruct((n, hid), jnp.float32),
            jax.ShapeDtypeStruct((n, c), jnp.float32),
        ],
        compiler_params=pltpu.CompilerParams(
            dimension_semantics=("arbitrary",)),
    )(adj, h1, b1r, W2)

    out = pl.pallas_call(
        _layer2_body,
        grid=grid,
        in_specs=[
            pl.BlockSpec((_BI, n), lambda i: (i, 0)),
            pl.BlockSpec((n, c), lambda i: (0, 0)),
            pl.BlockSpec((1, c), lambda i: (0, 0)),
        ],
        out_specs=pl.BlockSpec((_BI, c), lambda i: (i, 0)),
        out_shape=jax.ShapeDtypeStruct((n, c), jnp.float32),
        compiler_params=pltpu.CompilerParams(
            dimension_semantics=("arbitrary",)),
    )(adj, g2, b2r)

    return (x1, out)
